# Initial kernel scaffold; baseline (speedup 1.0000x reference)
import jax
import jax.numpy as jnp
from jax.experimental import pallas as pl


def kernel(X, v_idx, e_idx, W1_v2e, b1_v2e, W1_e2v, b1_e2v, W2_v2e, b2_v2e, W2_e2v, b2_e2v):
    raise NotImplementedError("write your pallas kernel here")


# speedup vs baseline: 2.9879x; 2.9879x over previous
import functools

import jax
import jax.numpy as jnp
from jax import lax
from jax.experimental import pallas as pl
from jax.experimental.pallas import tpu as pltpu
from jax.experimental.pallas import tpu_sc as plsc

NV, NE, NNZ = 10000, 5000, 320000
NVP, NEP = 10240, 5120
NNZP = 327680
NROW, B = 2560, 128
NC, NS = 2, 16
NW = NC * NS
RPW = NROW // NW
RCH = 16
NCH = RPW // RCH
NBUF = 2
CW = 16
F32 = jnp.float32


def _make_seg(ns, ndp, c, with_counts):
    mesh = plsc.VectorSubcoreMesh(core_axis_name="c", subcore_axis_name="s")
    out_type = [jax.ShapeDtypeStruct((NC, ndp, c), F32)]
    scratch = [
        pltpu.VMEM_SHARED((ndp, c), F32),
        pltpu.VMEM((RCH, B), jnp.int32),
        pltpu.VMEM((RCH, B), jnp.int32),
        pltpu.VMEM((NBUF, B, c), F32),
    ]
    if with_counts:
        out_type += [jax.ShapeDtypeStruct((NW, NEP), F32),
                     jax.ShapeDtypeStruct((NW, NVP), F32)]
        scratch += [pltpu.VMEM((RCH, B), jnp.int32),
                    pltpu.VMEM((NEP,), F32),
                    pltpu.VMEM((NVP,), F32)]
    scratch += [pltpu.SemaphoreType.DMA for _ in range(NBUF)]

    def body(*refs):
        cv2 = None
        if with_counts:
            (src, gidx, sidx, cidx, zc, z1d,
             p_out, ce_out, cv_out,
             acc, gv, sv, rows, cv2, ce_loc, cv_loc) = refs[:-NBUF]
        else:
            (src, gidx, sidx, zc,
             p_out,
             acc, gv, sv, rows) = refs[:-NBUF]
        sems = refs[-NBUF:]
        ci = lax.axis_index("c")
        si = lax.axis_index("s")
        wid = si * NC + ci
        rsub = ndp // NS
        srow = pl.multiple_of(si * rsub, 8)
        PZ = 64
        stg = rows.at[0].at[pl.ds(0, PZ)]
        pltpu.sync_copy(zc.at[pl.ds(0, PZ)], stg)
        for t in range(rsub // PZ):
            pltpu.sync_copy(stg, acc.at[pl.ds(
                pl.multiple_of(srow + t * PZ, 8), PZ)])
        if with_counts:
            pltpu.sync_copy(z1d.at[pl.ds(0, NEP)], ce_loc)
            pltpu.sync_copy(z1d, cv_loc)
        plsc.subcore_barrier()
        ones16 = jnp.ones((16,), F32)

        def chunk(ch, carry):
            base = pl.multiple_of(wid * RPW + ch * RCH, 8)
            pltpu.sync_copy(gidx.at[pl.ds(base, RCH)], gv)
            pltpu.sync_copy(sidx.at[pl.ds(base, RCH)], sv)
            if with_counts:
                pltpu.sync_copy(cidx.at[pl.ds(base, RCH)], cv2)
            descs = [pltpu.async_copy(src.at[gv.at[b]], rows.at[b], sems[b])
                     for b in range(NBUF)]
            for j in range(RCH):
                b = j % NBUF
                descs[b].wait()
                pltpu.sync_copy(rows.at[b], acc.at[sv.at[j]], add=True)
                if with_counts:
                    for k in range(B // 16):
                        plsc.addupdate_scatter(
                            ce_loc, [sv[j, pl.ds(k * 16, 16)]], ones16)
                        plsc.addupdate_scatter(
                            cv_loc, [cv2[j, pl.ds(k * 16, 16)]], ones16)
                if j + NBUF < RCH:
                    descs[b] = pltpu.async_copy(src.at[gv.at[j + NBUF]],
                                                rows.at[b], sems[b])
            return carry

        lax.fori_loop(0, NCH, chunk, 0)
        plsc.subcore_barrier()
        for t in range(rsub // PZ):
            r0 = pl.multiple_of(srow + t * PZ, 8)
            pltpu.sync_copy(acc.at[pl.ds(r0, PZ)], stg)
            pltpu.sync_copy(stg, p_out.at[ci, pl.ds(r0, PZ)])
        if with_counts:
            pltpu.sync_copy(ce_loc, ce_out.at[wid])
            pltpu.sync_copy(cv_loc, cv_out.at[wid])

    return pl.kernel(body, mesh=mesh, out_type=out_type, scratch_types=scratch,
                     compiler_params=pltpu.CompilerParams(
                         needs_layout_passes=False))


_seg_a = _make_seg(NV, NEP, 128, True)
_seg_b = _make_seg(NE, NVP, 128, False)
_seg_c = _make_seg(NV, NEP, 128, False)
_seg_d = _make_seg(NE, NVP, 128, False)


def _tc1_body(x_ref, w1_ref, b1_ref, we_ref, o_ref):
    h = jnp.maximum(
        jnp.dot(x_ref[...], w1_ref[...], preferred_element_type=F32) + b1_ref[...],
        0.0)
    o_ref[...] = jnp.dot(h, we_ref[...], preferred_element_type=F32)


def _tc2_body(nd, p_ref, c_ref, o_ref):
    cnt = jnp.sum(c_ref[...], axis=0)[:, None]
    s = p_ref[0] + p_ref[1]
    o_ref[...] = (s / jnp.clip(cnt, 1.0, None))[:nd]


def _tc3_body(p_ref, c_ref, b1e_ref, w2_ref, b2_ref, we_ref, o_ref):
    cnt = jnp.sum(c_ref[...], axis=0)[:, None]
    x1 = ((p_ref[0] + p_ref[1]) / jnp.clip(cnt, 1.0, None)
          + jnp.where(cnt > 0, 1.0, 0.0) * b1e_ref[...])[:NV]
    h2 = jnp.maximum(
        jnp.dot(x1, w2_ref[...], preferred_element_type=F32) + b2_ref[...], 0.0)
    o_ref[...] = jnp.dot(h2, we_ref[...], preferred_element_type=F32)


def _tc5_body(p_ref, c_ref, b2e_ref, o_ref):
    cnt = jnp.sum(c_ref[...], axis=0)[:, None]
    y = ((p_ref[0] + p_ref[1]) / jnp.clip(cnt, 1.0, None)
         + jnp.where(cnt > 0, 1.0, 0.0) * b2e_ref[...])
    o_ref[...] = y[:NV, :40]


def kernel(X, v_idx, e_idx, W1_v2e, b1_v2e, W1_e2v, b1_e2v,
           W2_v2e, b2_v2e, W2_e2v, b2_e2v):
    npad = NNZP - NNZ
    vi_g = jnp.concatenate([v_idx, jnp.zeros((npad,), jnp.int32)]).reshape(NROW, B)
    vi_s = jnp.concatenate([v_idx, jnp.full((npad,), NVP - 1, jnp.int32)]).reshape(NROW, B)
    ei_g = jnp.concatenate([e_idx, jnp.zeros((npad,), jnp.int32)]).reshape(NROW, B)
    ei_s = jnp.concatenate([e_idx, jnp.full((npad,), NEP - 1, jnp.int32)]).reshape(NROW, B)
    z128 = jnp.zeros((640, 128), F32)
    z1d = jnp.zeros((NVP,), F32)
    w2p = jnp.zeros((128, 128), F32).at[:, :40].set(W2_v2e)
    b2p = jnp.zeros((1, 128), F32).at[0, :40].set(b2_v2e)
    w2ep = jnp.zeros((128, 128), F32).at[:40, :40].set(W2_e2v)
    b2ep = jnp.zeros((1, 128), F32).at[0, :40].set(b2_e2v)

    G = pl.pallas_call(
        _tc1_body, out_shape=jax.ShapeDtypeStruct((NV, 128), F32),
    )(X, W1_v2e, b1_v2e.reshape(1, 128), W1_e2v)

    Pe, CE, CV = _seg_a(G, vi_g, ei_s, vi_s, z128, z1d)

    S = pl.pallas_call(
        functools.partial(_tc2_body, NE),
        out_shape=jax.ShapeDtypeStruct((NE, 128), F32),
    )(Pe, CE)

    Pv, = _seg_b(S, ei_g, vi_s, z128)

    G2 = pl.pallas_call(
        _tc3_body, out_shape=jax.ShapeDtypeStruct((NV, 128), F32),
    )(Pv, CV, b1_e2v.reshape(1, 128), w2p, b2p, w2ep)

    Pe2, = _seg_c(G2, vi_g, ei_s, z128)

    S2 = pl.pallas_call(
        functools.partial(_tc2_body, NE),
        out_shape=jax.ShapeDtypeStruct((NE, 128), F32),
    )(Pe2, CE)

    Pv2, = _seg_d(S2, ei_g, vi_s, z128)

    out = pl.pallas_call(
        _tc5_body, out_shape=jax.ShapeDtypeStruct((NV, 40), F32),
    )(Pv2, CV, b2ep)
    return out


# speedup vs baseline: 3.4467x; 1.1536x over previous
import functools

import jax
import jax.numpy as jnp
from jax import lax
from jax.experimental import pallas as pl
from jax.experimental.pallas import tpu as pltpu
from jax.experimental.pallas import tpu_sc as plsc

NV, NE, NNZ = 10000, 5000, 320000
NVP, NEP = 10240, 5120
NNZP = 327680
NROW, B = 5120, 64
NC, NS = 2, 16
NW = NC * NS
RPW = NROW // NW
RCH = 16
NCH = RPW // RCH
NBUF = 4
KAH = 2
CW = 16
F32 = jnp.float32


def _make_seg(ns, ndp, c, with_counts):
    mesh = plsc.VectorSubcoreMesh(core_axis_name="c", subcore_axis_name="s")
    out_type = [jax.ShapeDtypeStruct((NC, ndp, c), F32)]
    scratch = [
        pltpu.VMEM_SHARED((ndp, c), F32),
        pltpu.VMEM((RCH, B), jnp.int32),
        pltpu.VMEM((RCH, B), jnp.int32),
        pltpu.VMEM((NBUF, B, c), F32),
    ]
    if with_counts:
        out_type += [jax.ShapeDtypeStruct((NW, NEP), F32),
                     jax.ShapeDtypeStruct((NW, NVP), F32)]
        scratch += [pltpu.VMEM((RCH, B), jnp.int32),
                    pltpu.VMEM((NEP,), F32),
                    pltpu.VMEM((NVP,), F32)]
    scratch += [pltpu.SemaphoreType.DMA for _ in range(2 * NBUF)]

    def body(*refs):
        cv2 = None
        if with_counts:
            (src, gidx, sidx, cidx, zc, z1d,
             p_out, ce_out, cv_out,
             acc, gv, sv, rows, cv2, ce_loc, cv_loc) = refs[:-2 * NBUF]
        else:
            (src, gidx, sidx, zc,
             p_out,
             acc, gv, sv, rows) = refs[:-2 * NBUF]
        gsems = refs[-2 * NBUF:-NBUF]
        ssems = refs[-NBUF:]
        ci = lax.axis_index("c")
        si = lax.axis_index("s")
        wid = si * NC + ci
        rsub = ndp // NS
        srow = pl.multiple_of(si * rsub, 8)
        PZ = 64
        stg = rows.at[0].at[pl.ds(0, PZ)]
        pltpu.sync_copy(zc.at[pl.ds(0, PZ)], stg)
        for t in range(rsub // PZ):
            pltpu.sync_copy(stg, acc.at[pl.ds(
                pl.multiple_of(srow + t * PZ, 8), PZ)])
        if with_counts:
            pltpu.sync_copy(z1d.at[pl.ds(0, NEP)], ce_loc)
            pltpu.sync_copy(z1d, cv_loc)
        plsc.subcore_barrier()
        ones16 = jnp.ones((16,), F32)

        def chunk(ch, carry):
            base = pl.multiple_of(wid * RPW + ch * RCH, 8)
            pltpu.sync_copy(gidx.at[pl.ds(base, RCH)], gv)
            pltpu.sync_copy(sidx.at[pl.ds(base, RCH)], sv)
            if with_counts:
                pltpu.sync_copy(cidx.at[pl.ds(base, RCH)], cv2)
            gdesc = [None] * NBUF
            sdesc = [None] * NBUF
            for b in range(KAH):
                gdesc[b] = pltpu.async_copy(src.at[gv.at[b]], rows.at[b],
                                            gsems[b])
            for j in range(RCH):
                bj = j % NBUF
                if j + KAH < RCH:
                    bt = (j + KAH) % NBUF
                    if j - KAH >= 0:
                        sdesc[bt].wait()
                    gdesc[bt] = pltpu.async_copy(src.at[gv.at[j + KAH]],
                                                 rows.at[bt], gsems[bt])
                gdesc[bj].wait()
                sdesc[bj] = pltpu.async_copy(rows.at[bj], acc.at[sv.at[j]],
                                             ssems[bj], add=True)
                if with_counts:
                    for k in range(B // 16):
                        plsc.addupdate_scatter(
                            ce_loc, [sv[j, pl.ds(k * 16, 16)]], ones16)
                        plsc.addupdate_scatter(
                            cv_loc, [cv2[j, pl.ds(k * 16, 16)]], ones16)
            for b in range(NBUF):
                sdesc[b].wait()
            return carry

        lax.fori_loop(0, NCH, chunk, 0)
        plsc.subcore_barrier()
        for t in range(rsub // PZ):
            r0 = pl.multiple_of(srow + t * PZ, 8)
            pltpu.sync_copy(acc.at[pl.ds(r0, PZ)], stg)
            pltpu.sync_copy(stg, p_out.at[ci, pl.ds(r0, PZ)])
        if with_counts:
            pltpu.sync_copy(ce_loc, ce_out.at[wid])
            pltpu.sync_copy(cv_loc, cv_out.at[wid])

    return pl.kernel(body, mesh=mesh, out_type=out_type, scratch_types=scratch,
                     compiler_params=pltpu.CompilerParams(
                         needs_layout_passes=False))


_seg_a = _make_seg(NV, NEP, 128, True)
_seg_b = _make_seg(NE, NVP, 128, False)
_seg_c = _make_seg(NV, NEP, 128, False)
_seg_d = _make_seg(NE, NVP, 128, False)


def _tc1_body(x_ref, w1_ref, b1_ref, we_ref, o_ref):
    h = jnp.maximum(
        jnp.dot(x_ref[...], w1_ref[...], preferred_element_type=F32) + b1_ref[...],
        0.0)
    o_ref[...] = jnp.dot(h, we_ref[...], preferred_element_type=F32)


def _tc2_body(nd, p_ref, c_ref, o_ref):
    cnt = jnp.sum(c_ref[...], axis=0)[:, None]
    s = p_ref[0] + p_ref[1]
    o_ref[...] = (s / jnp.clip(cnt, 1.0, None))[:nd]


def _tc3_body(p_ref, c_ref, b1e_ref, w2_ref, b2_ref, we_ref, o_ref):
    cnt = jnp.sum(c_ref[...], axis=0)[:, None]
    x1 = ((p_ref[0] + p_ref[1]) / jnp.clip(cnt, 1.0, None)
          + jnp.where(cnt > 0, 1.0, 0.0) * b1e_ref[...])[:NV]
    h2 = jnp.maximum(
        jnp.dot(x1, w2_ref[...], preferred_element_type=F32) + b2_ref[...], 0.0)
    o_ref[...] = jnp.dot(h2, we_ref[...], preferred_element_type=F32)


def _tc5_body(p_ref, c_ref, b2e_ref, o_ref):
    cnt = jnp.sum(c_ref[...], axis=0)[:, None]
    y = ((p_ref[0] + p_ref[1]) / jnp.clip(cnt, 1.0, None)
         + jnp.where(cnt > 0, 1.0, 0.0) * b2e_ref[...])
    o_ref[...] = y[:NV, :40]


def kernel(X, v_idx, e_idx, W1_v2e, b1_v2e, W1_e2v, b1_e2v,
           W2_v2e, b2_v2e, W2_e2v, b2_e2v):
    npad = NNZP - NNZ
    vi_g = jnp.concatenate([v_idx, jnp.zeros((npad,), jnp.int32)]).reshape(NROW, B)
    vi_s = jnp.concatenate([v_idx, jnp.full((npad,), NVP - 1, jnp.int32)]).reshape(NROW, B)
    ei_g = jnp.concatenate([e_idx, jnp.zeros((npad,), jnp.int32)]).reshape(NROW, B)
    ei_s = jnp.concatenate([e_idx, jnp.full((npad,), NEP - 1, jnp.int32)]).reshape(NROW, B)
    z128 = jnp.zeros((640, 128), F32)
    z1d = jnp.zeros((NVP,), F32)
    w2p = jnp.zeros((128, 128), F32).at[:, :40].set(W2_v2e)
    b2p = jnp.zeros((1, 128), F32).at[0, :40].set(b2_v2e)
    w2ep = jnp.zeros((128, 128), F32).at[:40, :40].set(W2_e2v)
    b2ep = jnp.zeros((1, 128), F32).at[0, :40].set(b2_e2v)

    G = pl.pallas_call(
        _tc1_body, out_shape=jax.ShapeDtypeStruct((NV, 128), F32),
    )(X, W1_v2e, b1_v2e.reshape(1, 128), W1_e2v)

    Pe, CE, CV = _seg_a(G, vi_g, ei_s, vi_s, z128, z1d)

    S = pl.pallas_call(
        functools.partial(_tc2_body, NE),
        out_shape=jax.ShapeDtypeStruct((NE, 128), F32),
    )(Pe, CE)

    Pv, = _seg_b(S, ei_g, vi_s, z128)

    G2 = pl.pallas_call(
        _tc3_body, out_shape=jax.ShapeDtypeStruct((NV, 128), F32),
    )(Pv, CV, b1_e2v.reshape(1, 128), w2p, b2p, w2ep)

    Pe2, = _seg_c(G2, vi_g, ei_s, z128)

    S2 = pl.pallas_call(
        functools.partial(_tc2_body, NE),
        out_shape=jax.ShapeDtypeStruct((NE, 128), F32),
    )(Pe2, CE)

    Pv2, = _seg_d(S2, ei_g, vi_s, z128)

    out = pl.pallas_call(
        _tc5_body, out_shape=jax.ShapeDtypeStruct((NV, 40), F32),
    )(Pv2, CV, b2ep)
    return out


# speedup vs baseline: 3.8761x; 1.1246x over previous
import functools

import jax
import jax.numpy as jnp
from jax import lax
from jax.experimental import pallas as pl
from jax.experimental.pallas import tpu as pltpu
from jax.experimental.pallas import tpu_sc as plsc

NV, NE, NNZ = 10000, 5000, 320000
NVP, NEP = 10240, 5120
NNZP = 327680
NROW, B = 5120, 64
NC, NS = 2, 16
NW = NC * NS
RPW = NROW // NW
RCH = 16
NCH = RPW // RCH
NBUF = 4
KAH = 2
CW = 16
F32 = jnp.float32


def _make_seg(ns, ndp, c, with_counts):
    mesh = plsc.VectorSubcoreMesh(core_axis_name="c", subcore_axis_name="s")
    out_type = [jax.ShapeDtypeStruct((NC, ndp, c), F32)]
    scratch = [
        pltpu.VMEM_SHARED((ndp, c), F32),
        pltpu.VMEM((RCH, B), jnp.int32),
        pltpu.VMEM((RCH, B), jnp.int32),
        pltpu.VMEM((NBUF, B, c), F32),
    ]
    if with_counts:
        out_type += [jax.ShapeDtypeStruct((NW, NEP), F32),
                     jax.ShapeDtypeStruct((NW, NVP), F32)]
        scratch += [pltpu.VMEM((RCH, B), jnp.int32),
                    pltpu.VMEM((NEP,), F32),
                    pltpu.VMEM((NVP,), F32)]
    scratch += [pltpu.SemaphoreType.DMA for _ in range(2 * NBUF)]

    def body(*refs):
        cv2 = None
        if with_counts:
            (src, gidx, sidx, cidx, zc, z1d,
             p_out, ce_out, cv_out,
             acc, gv, sv, rows, cv2, ce_loc, cv_loc) = refs[:-2 * NBUF]
        else:
            (src, gidx, sidx, zc,
             p_out,
             acc, gv, sv, rows) = refs[:-2 * NBUF]
        gsems = refs[-2 * NBUF:-NBUF]
        ssems = refs[-NBUF:]
        ci = lax.axis_index("c")
        si = lax.axis_index("s")
        wid = si * NC + ci
        rsub = ndp // NS
        srow = pl.multiple_of(si * rsub, 8)
        PZ = 64
        stg = rows.at[0].at[pl.ds(0, PZ)]
        pltpu.sync_copy(zc.at[pl.ds(0, PZ)], stg)
        for t in range(rsub // PZ):
            pltpu.sync_copy(stg, acc.at[pl.ds(
                pl.multiple_of(srow + t * PZ, 8), PZ)])
        if with_counts:
            pltpu.sync_copy(z1d.at[pl.ds(0, NEP)], ce_loc)
            pltpu.sync_copy(z1d, cv_loc)
        plsc.subcore_barrier()
        ones16 = jnp.ones((16,), F32)

        def chunk(ch, carry):
            base = pl.multiple_of(wid * RPW + ch * RCH, 8)
            pltpu.sync_copy(gidx.at[pl.ds(base, RCH)], gv)
            pltpu.sync_copy(sidx.at[pl.ds(base, RCH)], sv)
            if with_counts:
                pltpu.sync_copy(cidx.at[pl.ds(base, RCH)], cv2)
            gdesc = [None] * NBUF
            sdesc = [None] * NBUF
            for b in range(KAH):
                gdesc[b] = pltpu.async_copy(src.at[gv.at[b]], rows.at[b],
                                            gsems[b])
            for j in range(RCH):
                bj = j % NBUF
                if j + KAH < RCH:
                    bt = (j + KAH) % NBUF
                    if j - KAH >= 0:
                        sdesc[bt].wait()
                    gdesc[bt] = pltpu.async_copy(src.at[gv.at[j + KAH]],
                                                 rows.at[bt], gsems[bt])
                gdesc[bj].wait()
                sdesc[bj] = pltpu.async_copy(rows.at[bj], acc.at[sv.at[j]],
                                             ssems[bj], add=True)
                if with_counts:
                    for k in range(B // 16):
                        plsc.addupdate_scatter(
                            ce_loc, [sv[j, pl.ds(k * 16, 16)]], ones16)
                        plsc.addupdate_scatter(
                            cv_loc, [cv2[j, pl.ds(k * 16, 16)]], ones16)
            for b in range(NBUF):
                sdesc[b].wait()
            return carry

        lax.fori_loop(0, NCH, chunk, 0)
        plsc.subcore_barrier()
        for t in range(rsub // PZ):
            r0 = pl.multiple_of(srow + t * PZ, 8)
            pltpu.sync_copy(acc.at[pl.ds(r0, PZ)], stg)
            pltpu.sync_copy(stg, p_out.at[ci, pl.ds(r0, PZ)])
        if with_counts:
            pltpu.sync_copy(ce_loc, ce_out.at[wid])
            pltpu.sync_copy(cv_loc, cv_out.at[wid])

    return pl.kernel(body, mesh=mesh, out_type=out_type, scratch_types=scratch,
                     compiler_params=pltpu.CompilerParams(
                         needs_layout_passes=False,
                         use_tc_tiling_on_sc=False if c != 128 else None))


_seg_a = _make_seg(NV, NEP, 128, True)
_seg_b = _make_seg(NE, NVP, 128, False)
_seg_c = _make_seg(NV, NEP, 64, False)
_seg_d = _make_seg(NE, NVP, 64, False)


def _tc1_body(x_ref, w1_ref, b1_ref, we_ref, o_ref):
    h = jnp.maximum(
        jnp.dot(x_ref[...], w1_ref[...], preferred_element_type=F32) + b1_ref[...],
        0.0)
    o_ref[...] = jnp.dot(h, we_ref[...], preferred_element_type=F32)


def _tc2_body(nd, p_ref, c_ref, o_ref):
    cnt = jnp.sum(c_ref[...], axis=0)[:, None]
    s = p_ref[0] + p_ref[1]
    o_ref[...] = (s / jnp.clip(cnt, 1.0, None))[:nd]


def _tc3_body(p_ref, c_ref, b1e_ref, w2_ref, b2_ref, we_ref, o_ref):
    cnt = jnp.sum(c_ref[...], axis=0)[:, None]
    x1 = ((p_ref[0] + p_ref[1]) / jnp.clip(cnt, 1.0, None)
          + jnp.where(cnt > 0, 1.0, 0.0) * b1e_ref[...])[:NV]
    h2 = jnp.maximum(
        jnp.dot(x1, w2_ref[...], preferred_element_type=F32) + b2_ref[...], 0.0)
    o_ref[...] = jnp.dot(h2, we_ref[...], preferred_element_type=F32)


def _tc5_body(p_ref, c_ref, b2e_ref, o_ref):
    cnt = jnp.sum(c_ref[...], axis=0)[:, None]
    y = ((p_ref[0] + p_ref[1]) / jnp.clip(cnt, 1.0, None)
         + jnp.where(cnt > 0, 1.0, 0.0) * b2e_ref[...])
    o_ref[...] = y[:NV, :40]


def kernel(X, v_idx, e_idx, W1_v2e, b1_v2e, W1_e2v, b1_e2v,
           W2_v2e, b2_v2e, W2_e2v, b2_e2v):
    npad = NNZP - NNZ
    vi_g = jnp.concatenate([v_idx, jnp.zeros((npad,), jnp.int32)]).reshape(NROW, B)
    vi_s = jnp.concatenate([v_idx, jnp.full((npad,), NVP - 1, jnp.int32)]).reshape(NROW, B)
    ei_g = jnp.concatenate([e_idx, jnp.zeros((npad,), jnp.int32)]).reshape(NROW, B)
    ei_s = jnp.concatenate([e_idx, jnp.full((npad,), NEP - 1, jnp.int32)]).reshape(NROW, B)
    z128 = jnp.zeros((640, 128), F32)
    z64 = jnp.zeros((640, 64), F32)
    z1d = jnp.zeros((NVP,), F32)
    w2p = jnp.zeros((128, 64), F32).at[:, :40].set(W2_v2e)
    b2p = jnp.zeros((1, 64), F32).at[0, :40].set(b2_v2e)
    w2ep = jnp.zeros((64, 64), F32).at[:40, :40].set(W2_e2v)
    b2ep = jnp.zeros((1, 64), F32).at[0, :40].set(b2_e2v)

    G = pl.pallas_call(
        _tc1_body, out_shape=jax.ShapeDtypeStruct((NV, 128), F32),
    )(X, W1_v2e, b1_v2e.reshape(1, 128), W1_e2v)

    Pe, CE, CV = _seg_a(G, vi_g, ei_s, vi_s, z128, z1d)

    S = pl.pallas_call(
        functools.partial(_tc2_body, NE),
        out_shape=jax.ShapeDtypeStruct((NE, 128), F32),
    )(Pe, CE)

    Pv, = _seg_b(S, ei_g, vi_s, z128)

    G2 = pl.pallas_call(
        _tc3_body, out_shape=jax.ShapeDtypeStruct((NV, 64), F32),
    )(Pv, CV, b1_e2v.reshape(1, 128), w2p, b2p, w2ep)

    Pe2, = _seg_c(G2, vi_g, ei_s, z64)

    S2 = pl.pallas_call(
        functools.partial(_tc2_body, NE),
        out_shape=jax.ShapeDtypeStruct((NE, 64), F32),
    )(Pe2, CE)

    Pv2, = _seg_d(S2, ei_g, vi_s, z64)

    out = pl.pallas_call(
        _tc5_body, out_shape=jax.ShapeDtypeStruct((NV, 40), F32),
    )(Pv2, CV, b2ep)
    return out


# speedup vs baseline: 4.4418x; 1.1459x over previous
import functools

import jax
import jax.numpy as jnp
from jax import lax
from jax.experimental import pallas as pl
from jax.experimental.pallas import tpu as pltpu
from jax.experimental.pallas import tpu_sc as plsc

NV, NE, NNZ = 10000, 5000, 320000
NVP, NEP = 10240, 5120
NNZP = 327680
NC, NS = 2, 16
NW = NC * NS
RCH = 16
NBUF = 4
KAH = 2
CW = 16
F32 = jnp.float32


def _make_seg(ns, ndp, c, with_counts, nrow, B):
    RPW = nrow // NW
    NCH = RPW // RCH
    mesh = plsc.VectorSubcoreMesh(core_axis_name="c", subcore_axis_name="s")
    out_type = [jax.ShapeDtypeStruct((NC, ndp, c), F32)]
    scratch = [
        pltpu.VMEM_SHARED((ndp, c), F32),
        pltpu.VMEM((RCH, B), jnp.int32),
        pltpu.VMEM((RCH, B), jnp.int32),
        pltpu.VMEM((NBUF, B, c), F32),
    ]
    if with_counts:
        out_type += [jax.ShapeDtypeStruct((NW, NEP), F32),
                     jax.ShapeDtypeStruct((NW, NVP), F32)]
        scratch += [pltpu.VMEM((RCH, B), jnp.int32),
                    pltpu.VMEM((NEP,), F32),
                    pltpu.VMEM((NVP,), F32)]
    scratch += [pltpu.SemaphoreType.DMA for _ in range(2 * NBUF)]

    def body(*refs):
        cv2 = None
        if with_counts:
            (src, gidx, sidx, cidx, zc, z1d,
             p_out, ce_out, cv_out,
             acc, gv, sv, rows, cv2, ce_loc, cv_loc) = refs[:-2 * NBUF]
        else:
            (src, gidx, sidx, zc,
             p_out,
             acc, gv, sv, rows) = refs[:-2 * NBUF]
        gsems = refs[-2 * NBUF:-NBUF]
        ssems = refs[-NBUF:]
        ci = lax.axis_index("c")
        si = lax.axis_index("s")
        wid = si * NC + ci
        rsub = ndp // NS
        srow = pl.multiple_of(si * rsub, 8)
        PZ = 64
        stg = rows.at[0].at[pl.ds(0, PZ)]
        pltpu.sync_copy(zc.at[pl.ds(0, PZ)], stg)
        for t in range(rsub // PZ):
            pltpu.sync_copy(stg, acc.at[pl.ds(
                pl.multiple_of(srow + t * PZ, 8), PZ)])
        if with_counts:
            pltpu.sync_copy(z1d.at[pl.ds(0, NEP)], ce_loc)
            pltpu.sync_copy(z1d, cv_loc)
        plsc.subcore_barrier()
        ones16 = jnp.ones((16,), F32)

        def chunk(ch, carry):
            base = pl.multiple_of(wid * RPW + ch * RCH, 8)
            pltpu.sync_copy(gidx.at[pl.ds(base, RCH)], gv)
            pltpu.sync_copy(sidx.at[pl.ds(base, RCH)], sv)
            if with_counts:
                pltpu.sync_copy(cidx.at[pl.ds(base, RCH)], cv2)
            gdesc = [None] * NBUF
            sdesc = [None] * NBUF
            for b in range(KAH):
                gdesc[b] = pltpu.async_copy(src.at[gv.at[b]], rows.at[b],
                                            gsems[b])
            for j in range(RCH):
                bj = j % NBUF
                if j + KAH < RCH:
                    bt = (j + KAH) % NBUF
                    if j - KAH >= 0:
                        sdesc[bt].wait()
                    gdesc[bt] = pltpu.async_copy(src.at[gv.at[j + KAH]],
                                                 rows.at[bt], gsems[bt])
                gdesc[bj].wait()
                sdesc[bj] = pltpu.async_copy(rows.at[bj], acc.at[sv.at[j]],
                                             ssems[bj], add=True)
                if with_counts:
                    for k in range(B // 16):
                        plsc.addupdate_scatter(
                            ce_loc, [sv[j, pl.ds(k * 16, 16)]], ones16)
                        plsc.addupdate_scatter(
                            cv_loc, [cv2[j, pl.ds(k * 16, 16)]], ones16)
            for b in range(NBUF):
                sdesc[b].wait()
            return carry

        lax.fori_loop(0, NCH, chunk, 0)
        plsc.subcore_barrier()
        for t in range(rsub // PZ):
            r0 = pl.multiple_of(srow + t * PZ, 8)
            pltpu.sync_copy(acc.at[pl.ds(r0, PZ)], stg)
            pltpu.sync_copy(stg, p_out.at[ci, pl.ds(r0, PZ)])
        if with_counts:
            pltpu.sync_copy(ce_loc, ce_out.at[wid])
            pltpu.sync_copy(cv_loc, cv_out.at[wid])

    return pl.kernel(body, mesh=mesh, out_type=out_type, scratch_types=scratch,
                     compiler_params=pltpu.CompilerParams(
                         needs_layout_passes=False,
                         use_tc_tiling_on_sc=False if c != 128 else None))


_seg_a = _make_seg(NV, NEP, 128, True, 2560, 128)
_seg_b = _make_seg(NE, NVP, 128, False, 4096, 80)
_seg_c = _make_seg(NV, NEP, 64, False, 2560, 128)
_seg_d = _make_seg(NE, NVP, 64, False, 4096, 80)


def _tc1_body(x_ref, w1_ref, b1_ref, we_ref, o_ref):
    h = jnp.maximum(
        jnp.dot(x_ref[...], w1_ref[...], preferred_element_type=F32) + b1_ref[...],
        0.0)
    o_ref[...] = jnp.dot(h, we_ref[...], preferred_element_type=F32)


def _tc2_body(nd, p_ref, c_ref, o_ref):
    cnt = jnp.sum(c_ref[...], axis=0)[:, None]
    s = p_ref[0] + p_ref[1]
    o_ref[...] = (s / jnp.clip(cnt, 1.0, None))[:nd]


def _tc3_body(p_ref, c_ref, b1e_ref, w2_ref, b2_ref, we_ref, o_ref):
    cnt = jnp.sum(c_ref[...], axis=0)[:, None]
    x1 = ((p_ref[0] + p_ref[1]) / jnp.clip(cnt, 1.0, None)
          + jnp.where(cnt > 0, 1.0, 0.0) * b1e_ref[...])[:NV]
    h2 = jnp.maximum(
        jnp.dot(x1, w2_ref[...], preferred_element_type=F32) + b2_ref[...], 0.0)
    o_ref[...] = jnp.dot(h2, we_ref[...], preferred_element_type=F32)


def _tc5_body(p_ref, c_ref, b2e_ref, o_ref):
    cnt = jnp.sum(c_ref[...], axis=0)[:, None]
    y = ((p_ref[0] + p_ref[1]) / jnp.clip(cnt, 1.0, None)
         + jnp.where(cnt > 0, 1.0, 0.0) * b2e_ref[...])
    o_ref[...] = y[:NV, :40]


def kernel(X, v_idx, e_idx, W1_v2e, b1_v2e, W1_e2v, b1_e2v,
           W2_v2e, b2_v2e, W2_e2v, b2_e2v):
    npad = NNZP - NNZ
    vi_gp = jnp.concatenate([v_idx, jnp.zeros((npad,), jnp.int32)])
    vi_sp = jnp.concatenate([v_idx, jnp.full((npad,), NVP - 1, jnp.int32)])
    ei_gp = jnp.concatenate([e_idx, jnp.zeros((npad,), jnp.int32)])
    ei_sp = jnp.concatenate([e_idx, jnp.full((npad,), NEP - 1, jnp.int32)])
    vi_g = vi_gp.reshape(2560, 128)
    vi_s = vi_sp.reshape(2560, 128)
    ei_s = ei_sp.reshape(2560, 128)
    ei_g_b = ei_gp.reshape(4096, 80)
    vi_s_b = vi_sp.reshape(4096, 80)
    z128 = jnp.zeros((640, 128), F32)
    z64 = jnp.zeros((640, 64), F32)
    z1d = jnp.zeros((NVP,), F32)
    w2p = jnp.zeros((128, 64), F32).at[:, :40].set(W2_v2e)
    b2p = jnp.zeros((1, 64), F32).at[0, :40].set(b2_v2e)
    w2ep = jnp.zeros((64, 64), F32).at[:40, :40].set(W2_e2v)
    b2ep = jnp.zeros((1, 64), F32).at[0, :40].set(b2_e2v)

    G = pl.pallas_call(
        _tc1_body, out_shape=jax.ShapeDtypeStruct((NV, 128), F32),
    )(X, W1_v2e, b1_v2e.reshape(1, 128), W1_e2v)

    Pe, CE, CV = _seg_a(G, vi_g, ei_s, vi_s, z128, z1d)

    S = pl.pallas_call(
        functools.partial(_tc2_body, NE),
        out_shape=jax.ShapeDtypeStruct((NE, 128), F32),
    )(Pe, CE)

    Pv, = _seg_b(S, ei_g_b, vi_s_b, z128)

    G2 = pl.pallas_call(
        _tc3_body, out_shape=jax.ShapeDtypeStruct((NV, 64), F32),
    )(Pv, CV, b1_e2v.reshape(1, 128), w2p, b2p, w2ep)

    Pe2, = _seg_c(G2, vi_g, ei_s, z64)

    S2 = pl.pallas_call(
        functools.partial(_tc2_body, NE),
        out_shape=jax.ShapeDtypeStruct((NE, 64), F32),
    )(Pe2, CE)

    Pv2, = _seg_d(S2, ei_g_b, vi_s_b, z64)

    out = pl.pallas_call(
        _tc5_body, out_shape=jax.ShapeDtypeStruct((NV, 40), F32),
    )(Pv2, CV, b2ep)
    return out
